# baseline (speedup 1.0000x reference)
import functools

import jax
import jax.numpy as jnp
from jax import lax
from jax.experimental import pallas as pl
from jax.experimental.pallas import tpu as pltpu
from jax.experimental.pallas import tpu_sc as plsc

N = 100000
D = 128
H = 256
B = 1024
M = 4096

NC = 2
NS = 16
NW = NC * NS

BKN = 112
NFULL = 13
NBPW = 14
CHUNK = NBPW * BKN
HALF = NW * CHUNK
NPAD = 2 * HALF
TRASH = B + M
ACC_ROWS = 5248
RPS = ACC_ROWS // NS


PBN = 1792
PBH = HALF // PBN
PBLK = NPAD // PBN


def _gate_body(kofs, f_ref, sm_ref, smf_ref, mid_ref, waw_ref, baw_ref,
               wf_ref, ws_ref):
    f = f_ref[...]
    t = jnp.dot(f, waw_ref[...], preferred_element_type=jnp.float32)[:, 0:1]
    w = jax.nn.sigmoid(t + baw_ref[0, 0])
    sm = sm_ref[0, 0, :]
    smf = smf_ref[0, 0, :]
    mid = mid_ref[0, 0, :]
    row = ((pl.program_id(0) + kofs) * PBN
           + lax.broadcasted_iota(jnp.int32, (PBN, D), 0))
    valid = row < N
    wf_ref[...] = jnp.where(valid, f * (w * sm[:, None]), 0.0)
    ws_ref[...] = jnp.where(
        valid,
        f * (w * (smf * (mid > 0).astype(jnp.float32))[:, None]), 0.0)


def _gate_half(k, node_feats, sm, smf, mid, waw8, baw):
    return pl.pallas_call(
        functools.partial(_gate_body, k * PBH),
        grid=(PBH,),
        in_specs=[
            pl.BlockSpec((PBN, D), lambda i: (i + k * PBH, 0)),
            pl.BlockSpec((1, 1, PBN), lambda i: (i + k * PBH, 0, 0)),
            pl.BlockSpec((1, 1, PBN), lambda i: (i + k * PBH, 0, 0)),
            pl.BlockSpec((1, 1, PBN), lambda i: (i + k * PBH, 0, 0)),
            pl.BlockSpec((D, 8), lambda i: (0, 0)),
            pl.BlockSpec((1, 1), lambda i: (0, 0)),
        ],
        out_specs=[
            pl.BlockSpec((PBN, D), lambda i: (i, 0)),
            pl.BlockSpec((PBN, D), lambda i: (i, 0)),
        ],
        out_shape=[
            jax.ShapeDtypeStruct((HALF, D), jnp.float32),
            jax.ShapeDtypeStruct((HALF, D), jnp.float32),
        ],
    )(node_feats, sm, smf, mid, waw8, baw)


def _sc_body(k, wf_h, ws_h, bidx_h, midx_h, zrows, parts,
             acc, fwf0, fwf1, fws0, fws1, bib, mib,
             sin0, sin1, ssc0, ssc1):
    core = lax.axis_index("c")
    sid = lax.axis_index("s")
    wid = core * NS + sid

    fwf = (fwf0, fwf1)
    fws = (fws0, fws1)
    sin = (sin0, sin1)
    ssc = (ssc0, ssc1)

    pltpu.sync_copy(zrows.at[pl.ds(sid * RPS, RPS), :],
                    acc.at[pl.ds(sid * RPS, RPS), :])

    wrow = k * NW + wid
    pltpu.sync_copy(bidx_h.at[wrow], bib)
    pltpu.sync_copy(midx_h.at[wrow], mib)
    plsc.subcore_barrier()

    base = wid * CHUNK

    def off_of(bi):
        return pl.multiple_of(base + jnp.minimum(bi, NFULL) * BKN, 8)

    def issue_in(p, bi):
        off = off_of(bi)
        pltpu.async_copy(wf_h.at[pl.ds(off, BKN), :], fwf[p], sin[p])
        pltpu.async_copy(ws_h.at[pl.ds(off, BKN), :], fws[p], sin[p])

    def wait_in(p, bi):
        off = off_of(bi)
        pltpu.make_async_copy(wf_h.at[pl.ds(off, BKN), :], fwf[p], sin[p]).wait()
        pltpu.make_async_copy(ws_h.at[pl.ds(off, BKN), :], fws[p], sin[p]).wait()

    def issue_sc(p, bi):
        bj = jnp.minimum(bi, NFULL)
        pltpu.async_copy(fwf[p], acc.at[bib.at[bj]], ssc[p], add=True)
        pltpu.async_copy(fws[p], acc.at[mib.at[bj]], ssc[p], add=True)

    def wait_sc(p, bi):
        bj = jnp.minimum(bi, NFULL)
        pltpu.make_async_copy(fwf[p], acc.at[bib.at[bj]], ssc[p]).wait()
        pltpu.make_async_copy(fws[p], acc.at[mib.at[bj]], ssc[p]).wait()

    issue_in(0, jnp.int32(0))

    def body(i, carry):
        for p in (0, 1):
            bi = 2 * i + p
            wait_in(p, bi)

            @pl.when(bi >= 1)
            def _():
                wait_sc(1 - p, bi - 1)

            issue_in(1 - p, bi + 1)
            issue_sc(p, bi)
        return carry

    lax.fori_loop(0, (NFULL + 1) // 2, body, jnp.int32(0))
    wait_in(0, jnp.int32(NFULL + 1))
    wait_sc(1, jnp.int32(NFULL))

    plsc.subcore_barrier()
    pltpu.sync_copy(acc.at[pl.ds(sid * RPS, RPS), :],
                    parts.at[core, pl.ds(sid * RPS, RPS), :])


def _sc_pool(k, wf, ws, bidx, midx, zrows):
    return pl.kernel(
        functools.partial(_sc_body, k),
        out_type=jax.ShapeDtypeStruct((NC, ACC_ROWS, D), jnp.float32),
        mesh=plsc.VectorSubcoreMesh(core_axis_name="c", subcore_axis_name="s"),
        scratch_types=[
            pltpu.VMEM_SHARED((ACC_ROWS, D), jnp.float32),
            pltpu.VMEM((BKN, D), jnp.float32),
            pltpu.VMEM((BKN, D), jnp.float32),
            pltpu.VMEM((BKN, D), jnp.float32),
            pltpu.VMEM((BKN, D), jnp.float32),
            pltpu.VMEM((NBPW, BKN), jnp.int32),
            pltpu.VMEM((NBPW, BKN), jnp.int32),
            pltpu.SemaphoreType.DMA,
            pltpu.SemaphoreType.DMA,
            pltpu.SemaphoreType.DMA,
            pltpu.SemaphoreType.DMA,
        ],
    )(wf, ws, bidx, midx, zrows)


def _mlp_body(p_ref, q_ref, wf_ref, bf_ref, w1_ref, b1_ref, w2_ref, b2_ref,
              x_ref, o_ref):
    x = (p_ref[0] + p_ref[1]) + (q_ref[0] + q_ref[1])
    x_ref[...] = x
    h0 = jnp.dot(x, wf_ref[...], preferred_element_type=jnp.float32) + bf_ref[...]
    h1 = jnp.maximum(
        jnp.dot(h0, w1_ref[...], preferred_element_type=jnp.float32) + b1_ref[...],
        0.0)
    o_ref[...] = jnp.dot(h1, w2_ref[...], preferred_element_type=jnp.float32) + b2_ref[...]


def kernel(node_feats, smask, smask_full, batch_ids, motif_ids,
           W_aw, b_aw, W_feat, b_feat, W1, b1, W2, b2):
    pad = NPAD - N
    sm = jnp.pad(smask, (0, pad)).reshape(PBLK, 1, PBN)
    smf = jnp.pad(smask_full, (0, pad)).reshape(PBLK, 1, PBN)
    mid = jnp.pad(motif_ids, (0, pad)).reshape(PBLK, 1, PBN)
    waw8 = jnp.pad(W_aw, ((0, 0), (0, 7)))
    baw = b_aw.reshape(1, 1)
    bidx = jnp.pad(batch_ids, (0, pad),
                   constant_values=TRASH).reshape(2 * NW, NBPW, BKN)
    midx = jnp.pad(motif_ids + (B - 1), (0, pad),
                   constant_values=TRASH).reshape(2 * NW, NBPW, BKN)
    zrows = jnp.zeros((ACC_ROWS, D), jnp.float32)

    wf1, ws1 = _gate_half(0, node_feats, sm, smf, mid, waw8, baw)
    parts1 = _sc_pool(0, wf1, ws1, bidx, midx, zrows)
    wf2, ws2 = _gate_half(1, node_feats, sm, smf, mid, waw8, baw)
    parts2 = _sc_pool(1, wf2, ws2, bidx, midx, zrows)

    xsum, out = pl.pallas_call(
        _mlp_body,
        grid=((B + M) // 512,),
        in_specs=[
            pl.BlockSpec((NC, 512, D), lambda i: (0, i, 0)),
            pl.BlockSpec((NC, 512, D), lambda i: (0, i, 0)),
            pl.BlockSpec((D, H), lambda i: (0, 0)),
            pl.BlockSpec((1, H), lambda i: (0, 0)),
            pl.BlockSpec((H, H), lambda i: (0, 0)),
            pl.BlockSpec((1, H), lambda i: (0, 0)),
            pl.BlockSpec((H, H // 2), lambda i: (0, 0)),
            pl.BlockSpec((1, H // 2), lambda i: (0, 0)),
        ],
        out_specs=[
            pl.BlockSpec((512, D), lambda i: (i, 0)),
            pl.BlockSpec((512, H // 2), lambda i: (i, 0)),
        ],
        out_shape=[
            jax.ShapeDtypeStruct((B + M, D), jnp.float32),
            jax.ShapeDtypeStruct((B + M, H // 2), jnp.float32),
        ],
    )(parts1, parts2, W_feat, b_feat.reshape(1, H), W1, b1.reshape(1, H),
      W2, b2.reshape(1, H // 2))

    return (xsum[:B], out[:B], out[B:])


# speedup vs baseline: 1.0143x; 1.0143x over previous
import functools

import jax
import jax.numpy as jnp
from jax import lax
from jax.experimental import pallas as pl
from jax.experimental.pallas import tpu as pltpu
from jax.experimental.pallas import tpu_sc as plsc

N = 100000
D = 128
H = 256
B = 1024
M = 4096

NC = 2
NS = 16
NW = NC * NS

BKN = 112
NFULL = 13
NBPW = 14
CHUNK = NBPW * BKN
HALF = NW * CHUNK
NPAD = 2 * HALF
TRASH = B + M
ACC_ROWS = 5248
RPS = ACC_ROWS // NS


PBN = 1792
PBH = HALF // PBN
PBLK = NPAD // PBN


def _gate_body(kofs, f_ref, sm_ref, smf_ref, mid_ref, waw_ref, baw_ref,
               wf_ref, ws_ref):
    f = f_ref[...]
    t = jnp.dot(f, waw_ref[...], preferred_element_type=jnp.float32)[:, 0:1]
    w = jax.nn.sigmoid(t + baw_ref[0, 0])
    sm = sm_ref[0, 0, :]
    smf = smf_ref[0, 0, :]
    mid = mid_ref[0, 0, :]
    row = ((pl.program_id(0) + kofs) * PBN
           + lax.broadcasted_iota(jnp.int32, (PBN, D), 0))
    valid = row < N
    wf_ref[...] = jnp.where(valid, f * (w * sm[:, None]), 0.0)
    ws_ref[...] = jnp.where(
        valid,
        f * (w * (smf * (mid > 0).astype(jnp.float32))[:, None]), 0.0)


def _gate_half(k, node_feats, sm, smf, mid, waw8, baw):
    return pl.pallas_call(
        functools.partial(_gate_body, k * PBH),
        grid=(PBH,),
        in_specs=[
            pl.BlockSpec((PBN, D), lambda i: (i + k * PBH, 0)),
            pl.BlockSpec((1, 1, PBN), lambda i: (i + k * PBH, 0, 0)),
            pl.BlockSpec((1, 1, PBN), lambda i: (i + k * PBH, 0, 0)),
            pl.BlockSpec((1, 1, PBN), lambda i: (i + k * PBH, 0, 0)),
            pl.BlockSpec((D, 8), lambda i: (0, 0)),
            pl.BlockSpec((1, 1), lambda i: (0, 0)),
        ],
        out_specs=[
            pl.BlockSpec((PBN, D), lambda i: (i, 0)),
            pl.BlockSpec((PBN, D), lambda i: (i, 0)),
        ],
        out_shape=[
            jax.ShapeDtypeStruct((HALF, D), jnp.float32),
            jax.ShapeDtypeStruct((HALF, D), jnp.float32),
        ],
    )(node_feats, sm, smf, mid, waw8, baw)


def _sc_body(k, wf_h, ws_h, bidx_h, midx_h, zrows, parts,
             acc, fwf0, fwf1, fws0, fws1, bib0, bib1, mib0, mib1,
             sin0, sin1, ssc0, ssc1):
    core = lax.axis_index("c")
    sid = lax.axis_index("s")
    wid = core * NS + sid

    fwf = (fwf0, fwf1)
    fws = (fws0, fws1)
    bib = (bib0, bib1)
    mib = (mib0, mib1)
    sin = (sin0, sin1)
    ssc = (ssc0, ssc1)

    pltpu.sync_copy(zrows.at[pl.ds(sid * RPS, RPS), :],
                    acc.at[pl.ds(sid * RPS, RPS), :])
    plsc.subcore_barrier()

    base = wid * CHUNK

    def off_of(bi):
        return pl.multiple_of(base + jnp.minimum(bi, NFULL) * BKN, 8)

    def issue_in(p, bi):
        off = off_of(bi)
        goff = pl.multiple_of(k * HALF + off, 8)
        pltpu.async_copy(wf_h.at[pl.ds(off, BKN), :], fwf[p], sin[p])
        pltpu.async_copy(ws_h.at[pl.ds(off, BKN), :], fws[p], sin[p])
        pltpu.async_copy(bidx_h.at[pl.ds(goff, BKN)], bib[p], sin[p])
        pltpu.async_copy(midx_h.at[pl.ds(goff, BKN)], mib[p], sin[p])

    def wait_in(p, bi):
        off = off_of(bi)
        goff = pl.multiple_of(k * HALF + off, 8)
        pltpu.make_async_copy(wf_h.at[pl.ds(off, BKN), :], fwf[p], sin[p]).wait()
        pltpu.make_async_copy(ws_h.at[pl.ds(off, BKN), :], fws[p], sin[p]).wait()
        pltpu.make_async_copy(bidx_h.at[pl.ds(goff, BKN)], bib[p], sin[p]).wait()
        pltpu.make_async_copy(midx_h.at[pl.ds(goff, BKN)], mib[p], sin[p]).wait()

    def issue_sc(p):
        pltpu.async_copy(fwf[p], acc.at[bib[p]], ssc[p], add=True)
        pltpu.async_copy(fws[p], acc.at[mib[p]], ssc[p], add=True)

    def wait_sc(p):
        pltpu.make_async_copy(fwf[p], acc.at[bib[p]], ssc[p]).wait()
        pltpu.make_async_copy(fws[p], acc.at[mib[p]], ssc[p]).wait()

    issue_in(0, jnp.int32(0))

    def body(i, carry):
        for p in (0, 1):
            bi = 2 * i + p
            wait_in(p, bi)

            @pl.when(bi >= 1)
            def _():
                wait_sc(1 - p)

            issue_in(1 - p, bi + 1)
            issue_sc(p)
        return carry

    lax.fori_loop(0, (NFULL + 1) // 2, body, jnp.int32(0))
    wait_in(0, jnp.int32(NFULL + 1))
    wait_sc(1)

    plsc.subcore_barrier()
    pltpu.sync_copy(acc.at[pl.ds(sid * RPS, RPS), :],
                    parts.at[core, pl.ds(sid * RPS, RPS), :])


def _sc_pool(k, wf, ws, bidx, midx, zrows):
    return pl.kernel(
        functools.partial(_sc_body, k),
        out_type=jax.ShapeDtypeStruct((NC, ACC_ROWS, D), jnp.float32),
        mesh=plsc.VectorSubcoreMesh(core_axis_name="c", subcore_axis_name="s"),
        scratch_types=[
            pltpu.VMEM_SHARED((ACC_ROWS, D), jnp.float32),
            pltpu.VMEM((BKN, D), jnp.float32),
            pltpu.VMEM((BKN, D), jnp.float32),
            pltpu.VMEM((BKN, D), jnp.float32),
            pltpu.VMEM((BKN, D), jnp.float32),
            pltpu.VMEM((BKN,), jnp.int32),
            pltpu.VMEM((BKN,), jnp.int32),
            pltpu.VMEM((BKN,), jnp.int32),
            pltpu.VMEM((BKN,), jnp.int32),
            pltpu.SemaphoreType.DMA,
            pltpu.SemaphoreType.DMA,
            pltpu.SemaphoreType.DMA,
            pltpu.SemaphoreType.DMA,
        ],
    )(wf, ws, bidx, midx, zrows)


def _mlp_body(p_ref, q_ref, wf_ref, bf_ref, w1_ref, b1_ref, w2_ref, b2_ref,
              x_ref, o_ref):
    x = (p_ref[0] + p_ref[1]) + (q_ref[0] + q_ref[1])
    x_ref[...] = x
    h0 = jnp.dot(x, wf_ref[...], preferred_element_type=jnp.float32) + bf_ref[...]
    h1 = jnp.maximum(
        jnp.dot(h0, w1_ref[...], preferred_element_type=jnp.float32) + b1_ref[...],
        0.0)
    o_ref[...] = jnp.dot(h1, w2_ref[...], preferred_element_type=jnp.float32) + b2_ref[...]


def kernel(node_feats, smask, smask_full, batch_ids, motif_ids,
           W_aw, b_aw, W_feat, b_feat, W1, b1, W2, b2):
    pad = NPAD - N
    sm = jnp.pad(smask, (0, pad)).reshape(PBLK, 1, PBN)
    smf = jnp.pad(smask_full, (0, pad)).reshape(PBLK, 1, PBN)
    mid = jnp.pad(motif_ids, (0, pad)).reshape(PBLK, 1, PBN)
    waw8 = jnp.pad(W_aw, ((0, 0), (0, 7)))
    baw = b_aw.reshape(1, 1)
    bidx = jnp.pad(batch_ids, (0, pad), constant_values=TRASH)
    midx = jnp.pad(motif_ids + (B - 1), (0, pad), constant_values=TRASH)
    zrows = jnp.zeros((ACC_ROWS, D), jnp.float32)

    wf1, ws1 = _gate_half(0, node_feats, sm, smf, mid, waw8, baw)
    parts1 = _sc_pool(0, wf1, ws1, bidx, midx, zrows)
    wf2, ws2 = _gate_half(1, node_feats, sm, smf, mid, waw8, baw)
    parts2 = _sc_pool(1, wf2, ws2, bidx, midx, zrows)

    xsum, out = pl.pallas_call(
        _mlp_body,
        grid=((B + M) // 512,),
        in_specs=[
            pl.BlockSpec((NC, 512, D), lambda i: (0, i, 0)),
            pl.BlockSpec((NC, 512, D), lambda i: (0, i, 0)),
            pl.BlockSpec((D, H), lambda i: (0, 0)),
            pl.BlockSpec((1, H), lambda i: (0, 0)),
            pl.BlockSpec((H, H), lambda i: (0, 0)),
            pl.BlockSpec((1, H), lambda i: (0, 0)),
            pl.BlockSpec((H, H // 2), lambda i: (0, 0)),
            pl.BlockSpec((1, H // 2), lambda i: (0, 0)),
        ],
        out_specs=[
            pl.BlockSpec((512, D), lambda i: (i, 0)),
            pl.BlockSpec((512, H // 2), lambda i: (i, 0)),
        ],
        out_shape=[
            jax.ShapeDtypeStruct((B + M, D), jnp.float32),
            jax.ShapeDtypeStruct((B + M, H // 2), jnp.float32),
        ],
    )(parts1, parts2, W_feat, b_feat.reshape(1, H), W1, b1.reshape(1, H),
      W2, b2.reshape(1, H // 2))

    return (xsum[:B], out[:B], out[B:])
